# baseline (speedup 1.0000x reference)
import functools

import jax
import jax.numpy as jnp
from jax import lax
from jax.experimental import pallas as pl
from jax.experimental.pallas import tpu as pltpu
from jax.experimental.pallas import tpu_sc as plsc

_N = 10000
_NPAD = 10240
_ZROW = _N
_NC, _NS = 2, 16
_NW = _NC * _NS
_CHUNK = 128
_RPT = _NPAD // _NS


def _sc_mesh():
    return plsc.VectorSubcoreMesh(core_axis_name="c", subcore_axis_name="s")


_SC_PARAMS = pltpu.CompilerParams(use_tc_tiling_on_sc=False)
_SC_PARAMS_NARROW = pltpu.CompilerParams(use_tc_tiling_on_sc=False,
                                         needs_layout_passes=False)


def _sc_degree(n_chunks):
    L = 16

    @functools.partial(
        pl.kernel,
        mesh=_sc_mesh(),
        out_type=jax.ShapeDtypeStruct((_NC, _NPAD), jnp.float32),
        scratch_types=[
            pltpu.VMEM((n_chunks, _CHUNK), jnp.int32),
            pltpu.VMEM((_CHUNK,), jnp.float32),
            pltpu.VMEM_SHARED((_NPAD,), jnp.float32),
        ],
        compiler_params=_SC_PARAMS_NARROW,
    )
    def k(dst_hbm, zeros_hbm, out_hbm, dst_v, ones_v, acc):
        c = lax.axis_index("c")
        s = lax.axis_index("s")
        wid = c * _NS + s
        pltpu.sync_copy(dst_hbm.at[wid], dst_v)
        one = lax.full((L,), 1.0, jnp.float32)
        for g in range(_CHUNK // L):
            ones_v[pl.ds(g * L, L)] = one
        pltpu.sync_copy(zeros_hbm.at[pl.ds(s * _RPT, _RPT)],
                        acc.at[pl.ds(s * _RPT, _RPT)])
        plsc.subcore_barrier()

        def body(i, carry):
            pltpu.sync_copy(ones_v, acc.at[dst_v.at[i]], add=True)
            return carry

        lax.fori_loop(0, n_chunks, body, 0)
        plsc.subcore_barrier()
        pltpu.sync_copy(acc.at[pl.ds(s * _RPT, _RPT)],
                        out_hbm.at[c, pl.ds(s * _RPT, _RPT)])

    return k


_NBUF = 4


def _sc_aggregate(d_feat, n_chunks):
    @functools.partial(
        pl.kernel,
        mesh=_sc_mesh(),
        out_type=jax.ShapeDtypeStruct((_NC, _NPAD, d_feat), jnp.float32),
        scratch_types=[
            pltpu.VMEM((n_chunks, _CHUNK), jnp.int32),
            pltpu.VMEM((n_chunks, _CHUNK), jnp.int32),
            pltpu.VMEM((_CHUNK, d_feat), jnp.float32),
            pltpu.VMEM_SHARED((_NPAD, d_feat), jnp.float32),
            pltpu.SemaphoreType.DMA,
        ],
        compiler_params=_SC_PARAMS,
    )
    def k(y_hbm, src_hbm, dst_hbm, zeros_hbm, out_hbm,
          src_v, dst_v, rows_v, acc, sem):
        c = lax.axis_index("c")
        s = lax.axis_index("s")
        wid = c * _NS + s
        pltpu.sync_copy(src_hbm.at[wid], src_v)
        pltpu.sync_copy(dst_hbm.at[wid], dst_v)
        pltpu.sync_copy(zeros_hbm.at[pl.ds(s * _RPT, _RPT)],
                        acc.at[pl.ds(s * _RPT, _RPT)])
        plsc.subcore_barrier()

        def body(i, carry):
            pltpu.async_copy(y_hbm.at[src_v.at[i]], rows_v, sem).wait()
            pltpu.sync_copy(rows_v, acc.at[dst_v.at[i]], add=True)
            return carry

        lax.fori_loop(0, n_chunks, body, 0)
        plsc.subcore_barrier()
        pltpu.sync_copy(acc.at[pl.ds(s * _RPT, _RPT)],
                        out_hbm.at[c, pl.ds(s * _RPT, _RPT)])

    return k


def _tc_pre(x_p, w1_p, degp_t):
    def body(x_ref, w_ref, dp_ref, y_ref, dinv_ref):
        dinv = lax.rsqrt(dp_ref[:, 0:1] + dp_ref[:, 1:2] + 1.0)
        xw = jnp.dot(x_ref[...], w_ref[...],
                     preferred_element_type=jnp.float32)
        y_ref[...] = xw * dinv
        dinv_ref[...] = dinv

    return pl.pallas_call(
        body,
        out_shape=[
            jax.ShapeDtypeStruct((_NPAD, w1_p.shape[1]), jnp.float32),
            jax.ShapeDtypeStruct((_NPAD, 1), jnp.float32),
        ],
    )(x_p, w1_p, degp_t)


def _tc_mid(p0, p1, y_prev, dinv, b_p, w_p):
    def body(p0_ref, p1_ref, yp_ref, dinv_ref, b_ref, w_ref, yn_ref):
        tot = p0_ref[...] + p1_ref[...] + yp_ref[...]
        h = jnp.tanh(dinv_ref[...] * tot + b_ref[...])
        rows = lax.broadcasted_iota(jnp.int32, (_NPAD, 1), 0)
        h = jnp.where(rows < _N, h, 0.0)
        yn_ref[...] = dinv_ref[...] * jnp.dot(
            h, w_ref[...], preferred_element_type=jnp.float32)

    return pl.pallas_call(
        body,
        out_shape=jax.ShapeDtypeStruct((_NPAD, w_p.shape[1]), jnp.float32),
    )(p0, p1, y_prev, dinv, b_p, w_p)


def _tc_final(p0, p1, y_prev, dinv, b_p, wc_p, bc_p):
    def body(p0_ref, p1_ref, yp_ref, dinv_ref, b_ref, wc_ref, bc_ref,
             h_ref, out_ref):
        tot = p0_ref[...] + p1_ref[...] + yp_ref[...]
        h = jnp.tanh(dinv_ref[...] * tot + b_ref[...])
        rows = lax.broadcasted_iota(jnp.int32, (_NPAD, 1), 0)
        h = jnp.where(rows < _N, h, 0.0)
        h_ref[...] = h
        out_ref[...] = jnp.dot(
            h, wc_ref[...], preferred_element_type=jnp.float32) + bc_ref[...]

    return pl.pallas_call(
        body,
        out_shape=[
            jax.ShapeDtypeStruct((_NPAD, b_p.shape[1]), jnp.float32),
            jax.ShapeDtypeStruct((_NPAD, wc_p.shape[1]), jnp.float32),
        ],
    )(p0, p1, y_prev, dinv, b_p, wc_p, bc_p)


def _pad2(a, rows, cols):
    return jnp.pad(a, ((0, rows - a.shape[0]), (0, cols - a.shape[1])))


def kernel(x, edge_index, W1, b1, W2, b2, W3, b3, Wc, bc):
    f32 = jnp.float32
    src = edge_index[0].astype(jnp.int32)
    dst = edge_index[1].astype(jnp.int32)
    n_edges = src.shape[0]
    n_chunks = -(-n_edges // (_NW * _CHUNK))
    n_chunks = -(-n_chunks // _NBUF) * _NBUF
    e_pad = _NW * _CHUNK * n_chunks
    pad = e_pad - n_edges

    src3 = jnp.concatenate(
        [src, jnp.full((pad,), _ZROW, jnp.int32)]).reshape(_NW, n_chunks, _CHUNK)
    dst3 = jnp.concatenate(
        [dst, jnp.full((pad,), _ZROW, jnp.int32)]).reshape(_NW, n_chunks, _CHUNK)

    z1 = jnp.zeros((_NPAD,), f32)
    z8 = jnp.zeros((_NPAD, 8), f32)
    z80 = jnp.zeros((_NPAD, 80), f32)

    x_p = _pad2(x, _NPAD, 128)
    w1_p = _pad2(W1, 128, 80)
    w2_p = _pad2(W2, 80, 8)
    w3_p = _pad2(W3, 8, 8)
    wc_p = _pad2(Wc, 8, 12)
    b1_p = jnp.pad(b1, (0, 80 - b1.shape[0]))[None, :]
    b2_p = jnp.pad(b2, (0, 8 - b2.shape[0]))[None, :]
    b3_p = jnp.pad(b3, (0, 8 - b3.shape[0]))[None, :]
    bc_p = bc[None, :]

    degp = _sc_degree(n_chunks)(dst3, z1)
    y1, dinv = _tc_pre(x_p, w1_p, degp.T)

    p1 = _sc_aggregate(80, n_chunks)(y1, src3, dst3, z80)
    y2 = _tc_mid(p1[0], p1[1], y1, dinv, b1_p, w2_p)

    p2 = _sc_aggregate(8, n_chunks)(y2, src3, dst3, z8)
    y3 = _tc_mid(p2[0], p2[1], y2, dinv, b2_p, w3_p)

    p3 = _sc_aggregate(8, n_chunks)(y3, src3, dst3, z8)
    h_p, out_p = _tc_final(p3[0], p3[1], y3, dinv, b3_p, wc_p, bc_p)

    return out_p[:_N], h_p[:_N, :4]


# speedup vs baseline: 1.0002x; 1.0002x over previous
import functools

import jax
import jax.numpy as jnp
from jax import lax
from jax.experimental import pallas as pl
from jax.experimental.pallas import tpu as pltpu
from jax.experimental.pallas import tpu_sc as plsc

_N = 10000
_NPAD = 10240
_ZROW = _N
_NC, _NS = 2, 16
_NW = _NC * _NS
_CHUNK = 128
_RPT = _NPAD // _NS


def _sc_mesh():
    return plsc.VectorSubcoreMesh(core_axis_name="c", subcore_axis_name="s")


_SC_PARAMS = pltpu.CompilerParams(use_tc_tiling_on_sc=False)
_SC_PARAMS_NARROW = pltpu.CompilerParams(use_tc_tiling_on_sc=False,
                                         needs_layout_passes=False)


def _sc_degree(n_chunks):
    L = 16

    @functools.partial(
        pl.kernel,
        mesh=_sc_mesh(),
        out_type=jax.ShapeDtypeStruct((_NC, _NPAD), jnp.float32),
        scratch_types=[
            pltpu.VMEM((n_chunks, _CHUNK), jnp.int32),
            pltpu.VMEM((_CHUNK,), jnp.float32),
            pltpu.VMEM_SHARED((_NPAD,), jnp.float32),
        ],
        compiler_params=_SC_PARAMS,
    )
    def k(dst_hbm, zeros_hbm, out_hbm, dst_v, ones_v, acc):
        c = lax.axis_index("c")
        s = lax.axis_index("s")
        wid = c * _NS + s
        pltpu.sync_copy(dst_hbm.at[wid], dst_v)
        one = lax.full((L,), 1.0, jnp.float32)
        for g in range(_CHUNK // L):
            ones_v[pl.ds(g * L, L)] = one
        pltpu.sync_copy(zeros_hbm.at[pl.ds(s * _RPT, _RPT)],
                        acc.at[pl.ds(s * _RPT, _RPT)])
        plsc.subcore_barrier()

        def body(i, carry):
            pltpu.sync_copy(ones_v, acc.at[dst_v.at[i]], add=True)
            return carry

        lax.fori_loop(0, n_chunks, body, 0)
        plsc.subcore_barrier()
        pltpu.sync_copy(acc.at[pl.ds(s * _RPT, _RPT)],
                        out_hbm.at[c, pl.ds(s * _RPT, _RPT)])

    return k


_NBUF = 4


def _sc_aggregate(d_feat, n_chunks):
    @functools.partial(
        pl.kernel,
        mesh=_sc_mesh(),
        out_type=jax.ShapeDtypeStruct((_NC, _NPAD, d_feat), jnp.float32),
        scratch_types=[
            pltpu.VMEM((n_chunks, _CHUNK), jnp.int32),
            pltpu.VMEM((n_chunks, _CHUNK), jnp.int32),
            pltpu.VMEM((_CHUNK, d_feat), jnp.float32),
            pltpu.VMEM_SHARED((_NPAD, d_feat), jnp.float32),
            pltpu.SemaphoreType.DMA,
        ],
        compiler_params=_SC_PARAMS,
    )
    def k(y_hbm, src_hbm, dst_hbm, zeros_hbm, out_hbm,
          src_v, dst_v, rows_v, acc, sem):
        c = lax.axis_index("c")
        s = lax.axis_index("s")
        wid = c * _NS + s
        pltpu.sync_copy(src_hbm.at[wid], src_v)
        pltpu.sync_copy(dst_hbm.at[wid], dst_v)
        pltpu.sync_copy(zeros_hbm.at[pl.ds(s * _RPT, _RPT)],
                        acc.at[pl.ds(s * _RPT, _RPT)])
        plsc.subcore_barrier()

        def body(i, carry):
            pltpu.async_copy(y_hbm.at[src_v.at[i]], rows_v, sem).wait()
            pltpu.sync_copy(rows_v, acc.at[dst_v.at[i]], add=True)
            return carry

        lax.fori_loop(0, n_chunks, body, 0)
        plsc.subcore_barrier()
        pltpu.sync_copy(acc.at[pl.ds(s * _RPT, _RPT)],
                        out_hbm.at[c, pl.ds(s * _RPT, _RPT)])

    return k


def _tc_pre(x_p, w1_p, degp_t):
    def body(x_ref, w_ref, dp_ref, y_ref, dinv_ref):
        dinv = lax.rsqrt(dp_ref[:, 0:1] + dp_ref[:, 1:2] + 1.0)
        xw = jnp.dot(x_ref[...], w_ref[...],
                     preferred_element_type=jnp.float32)
        y_ref[...] = xw * dinv
        dinv_ref[...] = dinv

    return pl.pallas_call(
        body,
        out_shape=[
            jax.ShapeDtypeStruct((_NPAD, w1_p.shape[1]), jnp.float32),
            jax.ShapeDtypeStruct((_NPAD, 1), jnp.float32),
        ],
    )(x_p, w1_p, degp_t)


def _tc_mid(p0, p1, y_prev, dinv, b_p, w_p):
    def body(p0_ref, p1_ref, yp_ref, dinv_ref, b_ref, w_ref, yn_ref):
        tot = p0_ref[...] + p1_ref[...] + yp_ref[...]
        h = jnp.tanh(dinv_ref[...] * tot + b_ref[...])
        rows = lax.broadcasted_iota(jnp.int32, (_NPAD, 1), 0)
        h = jnp.where(rows < _N, h, 0.0)
        yn_ref[...] = dinv_ref[...] * jnp.dot(
            h, w_ref[...], preferred_element_type=jnp.float32)

    return pl.pallas_call(
        body,
        out_shape=jax.ShapeDtypeStruct((_NPAD, w_p.shape[1]), jnp.float32),
    )(p0, p1, y_prev, dinv, b_p, w_p)


def _tc_final(p0, p1, y_prev, dinv, b_p, wc_p, bc_p):
    def body(p0_ref, p1_ref, yp_ref, dinv_ref, b_ref, wc_ref, bc_ref,
             h_ref, out_ref):
        tot = p0_ref[...] + p1_ref[...] + yp_ref[...]
        h = jnp.tanh(dinv_ref[...] * tot + b_ref[...])
        rows = lax.broadcasted_iota(jnp.int32, (_NPAD, 1), 0)
        h = jnp.where(rows < _N, h, 0.0)
        h_ref[...] = h
        out_ref[...] = jnp.dot(
            h, wc_ref[...], preferred_element_type=jnp.float32) + bc_ref[...]

    return pl.pallas_call(
        body,
        out_shape=[
            jax.ShapeDtypeStruct((_NPAD, b_p.shape[1]), jnp.float32),
            jax.ShapeDtypeStruct((_NPAD, wc_p.shape[1]), jnp.float32),
        ],
    )(p0, p1, y_prev, dinv, b_p, wc_p, bc_p)


def _pad2(a, rows, cols):
    return jnp.pad(a, ((0, rows - a.shape[0]), (0, cols - a.shape[1])))


def kernel(x, edge_index, W1, b1, W2, b2, W3, b3, Wc, bc):
    f32 = jnp.float32
    src = edge_index[0].astype(jnp.int32)
    dst = edge_index[1].astype(jnp.int32)
    n_edges = src.shape[0]
    n_chunks = -(-n_edges // (_NW * _CHUNK))
    n_chunks = -(-n_chunks // _NBUF) * _NBUF
    e_pad = _NW * _CHUNK * n_chunks
    pad = e_pad - n_edges

    src3 = jnp.concatenate(
        [src, jnp.full((pad,), _ZROW, jnp.int32)]).reshape(_NW, n_chunks, _CHUNK)
    dst3 = jnp.concatenate(
        [dst, jnp.full((pad,), _ZROW, jnp.int32)]).reshape(_NW, n_chunks, _CHUNK)

    z1 = jnp.zeros((_NPAD,), f32)
    z8 = jnp.zeros((_NPAD, 8), f32)
    z80 = jnp.zeros((_NPAD, 80), f32)

    x_p = _pad2(x, _NPAD, 128)
    w1_p = _pad2(W1, 128, 80)
    w2_p = _pad2(W2, 80, 8)
    w3_p = _pad2(W3, 8, 8)
    wc_p = _pad2(Wc, 8, 12)
    b1_p = jnp.pad(b1, (0, 80 - b1.shape[0]))[None, :]
    b2_p = jnp.pad(b2, (0, 8 - b2.shape[0]))[None, :]
    b3_p = jnp.pad(b3, (0, 8 - b3.shape[0]))[None, :]
    bc_p = bc[None, :]

    degp = _sc_degree(n_chunks)(dst3, z1)
    y1, dinv = _tc_pre(x_p, w1_p, degp.T)

    p1 = _sc_aggregate(80, n_chunks)(y1, src3, dst3, z80)
    y2 = _tc_mid(p1[0], p1[1], y1, dinv, b1_p, w2_p)

    p2 = _sc_aggregate(8, n_chunks)(y2, src3, dst3, z8)
    y3 = _tc_mid(p2[0], p2[1], y2, dinv, b2_p, w3_p)

    p3 = _sc_aggregate(8, n_chunks)(y3, src3, dst3, z8)
    h_p, out_p = _tc_final(p3[0], p3[1], y3, dinv, b3_p, wc_p, bc_p)

    return out_p[:_N], h_p[:_N, :4]


# speedup vs baseline: 1.7709x; 1.7705x over previous
import functools

import jax
import jax.numpy as jnp
from jax import lax
from jax.experimental import pallas as pl
from jax.experimental.pallas import tpu as pltpu
from jax.experimental.pallas import tpu_sc as plsc

_N = 10000
_NPAD = 10240
_ZROW = _N
_NC, _NS = 2, 16
_NW = _NC * _NS
_CHUNK = 128
_RPT = _NPAD // _NS


def _sc_mesh():
    return plsc.VectorSubcoreMesh(core_axis_name="c", subcore_axis_name="s")


_SC_PARAMS = pltpu.CompilerParams(use_tc_tiling_on_sc=False)
_SC_PARAMS_NARROW = pltpu.CompilerParams(use_tc_tiling_on_sc=False,
                                         needs_layout_passes=False)


def _sc_degree(n_chunks):
    L = 16

    @functools.partial(
        pl.kernel,
        mesh=_sc_mesh(),
        out_type=jax.ShapeDtypeStruct((_NC, _NPAD), jnp.float32),
        scratch_types=[
            pltpu.VMEM((n_chunks, _CHUNK), jnp.int32),
            pltpu.VMEM((_CHUNK,), jnp.float32),
            pltpu.VMEM_SHARED((_NPAD,), jnp.float32),
        ],
        compiler_params=_SC_PARAMS,
    )
    def k(dst_hbm, zeros_hbm, out_hbm, dst_v, ones_v, acc):
        c = lax.axis_index("c")
        s = lax.axis_index("s")
        wid = c * _NS + s
        pltpu.sync_copy(dst_hbm.at[wid], dst_v)
        one = lax.full((L,), 1.0, jnp.float32)
        for g in range(_CHUNK // L):
            ones_v[pl.ds(g * L, L)] = one
        pltpu.sync_copy(zeros_hbm.at[pl.ds(s * _RPT, _RPT)],
                        acc.at[pl.ds(s * _RPT, _RPT)])
        plsc.subcore_barrier()

        def body(i, carry):
            pltpu.sync_copy(ones_v, acc.at[dst_v.at[i]], add=True)
            return carry

        lax.fori_loop(0, n_chunks, body, 0)
        plsc.subcore_barrier()
        pltpu.sync_copy(acc.at[pl.ds(s * _RPT, _RPT)],
                        out_hbm.at[c, pl.ds(s * _RPT, _RPT)])

    return k


_NBUF = 4


def _sc_aggregate(d_feat, n_chunks):
    @functools.partial(
        pl.kernel,
        mesh=_sc_mesh(),
        out_type=jax.ShapeDtypeStruct((_NC, _NPAD, d_feat), jnp.float32),
        scratch_types=[
            pltpu.VMEM((n_chunks, _CHUNK), jnp.int32),
            pltpu.VMEM((n_chunks, _CHUNK), jnp.int32),
            pltpu.VMEM((_CHUNK, d_feat), jnp.float32),
            pltpu.VMEM_SHARED((_NPAD, d_feat), jnp.float32),
            pltpu.SemaphoreType.DMA,
        ],
        compiler_params=_SC_PARAMS,
    )
    def k(y_hbm, src_hbm, dst_hbm, zeros_hbm, out_hbm,
          src_v, dst_v, rows_v, acc, sem):
        c = lax.axis_index("c")
        s = lax.axis_index("s")
        wid = c * _NS + s
        pltpu.sync_copy(src_hbm.at[wid], src_v)
        pltpu.sync_copy(dst_hbm.at[wid], dst_v)
        pltpu.sync_copy(zeros_hbm.at[pl.ds(s * _RPT, _RPT)],
                        acc.at[pl.ds(s * _RPT, _RPT)])
        plsc.subcore_barrier()

        def body(i, carry):
            pltpu.async_copy(y_hbm.at[src_v.at[i]], rows_v, sem).wait()
            pltpu.sync_copy(rows_v, acc.at[dst_v.at[i]], add=True)
            return carry

        lax.fori_loop(0, n_chunks, body, 0)
        plsc.subcore_barrier()
        pltpu.sync_copy(acc.at[pl.ds(s * _RPT, _RPT)],
                        out_hbm.at[c, pl.ds(s * _RPT, _RPT)])

    return k


def _tc_pre(x_p, w1_p, degp_t):
    def body(x_ref, w_ref, dp_ref, y_ref, dinv_ref):
        dinv = lax.rsqrt(dp_ref[:, 0:1] + dp_ref[:, 1:2] + 1.0)
        xw = jnp.dot(x_ref[...], w_ref[...],
                     preferred_element_type=jnp.float32)
        y_ref[...] = xw * dinv
        dinv_ref[...] = dinv

    return pl.pallas_call(
        body,
        out_shape=[
            jax.ShapeDtypeStruct((_NPAD, w1_p.shape[1]), jnp.float32),
            jax.ShapeDtypeStruct((_NPAD, 1), jnp.float32),
        ],
    )(x_p, w1_p, degp_t)


def _tc_mid(p0, p1, y_prev, dinv, b_p, w_p):
    def body(p0_ref, p1_ref, yp_ref, dinv_ref, b_ref, w_ref, yn_ref):
        tot = p0_ref[...] + p1_ref[...] + yp_ref[...]
        h = jnp.tanh(dinv_ref[...] * tot + b_ref[...])
        rows = lax.broadcasted_iota(jnp.int32, (_NPAD, 1), 0)
        h = jnp.where(rows < _N, h, 0.0)
        yn_ref[...] = dinv_ref[...] * jnp.dot(
            h, w_ref[...], preferred_element_type=jnp.float32)

    return pl.pallas_call(
        body,
        out_shape=jax.ShapeDtypeStruct((_NPAD, w_p.shape[1]), jnp.float32),
    )(p0, p1, y_prev, dinv, b_p, w_p)


def _tc_final(p0, p1, y_prev, dinv, b_p, wc_p, bc_p):
    def body(p0_ref, p1_ref, yp_ref, dinv_ref, b_ref, wc_ref, bc_ref,
             h_ref, out_ref):
        tot = p0_ref[...] + p1_ref[...] + yp_ref[...]
        h = jnp.tanh(dinv_ref[...] * tot + b_ref[...])
        rows = lax.broadcasted_iota(jnp.int32, (_NPAD, 1), 0)
        h = jnp.where(rows < _N, h, 0.0)
        h_ref[...] = h
        out_ref[...] = jnp.dot(
            h, wc_ref[...], preferred_element_type=jnp.float32) + bc_ref[...]

    return pl.pallas_call(
        body,
        out_shape=[
            jax.ShapeDtypeStruct((_NPAD, b_p.shape[1]), jnp.float32),
            jax.ShapeDtypeStruct((_NPAD, wc_p.shape[1]), jnp.float32),
        ],
    )(p0, p1, y_prev, dinv, b_p, wc_p, bc_p)


def _pad2(a, rows, cols):
    return jnp.pad(a, ((0, rows - a.shape[0]), (0, cols - a.shape[1])))


def kernel(x, edge_index, W1, b1, W2, b2, W3, b3, Wc, bc):
    f32 = jnp.float32
    src = edge_index[0].astype(jnp.int32)
    dst = edge_index[1].astype(jnp.int32)
    n_edges = src.shape[0]
    n_chunks = -(-n_edges // (_NW * _CHUNK))
    e_pad = _NW * _CHUNK * n_chunks
    pad = e_pad - n_edges

    pad_dst = _N + jnp.arange(pad, dtype=jnp.int32) % (_NPAD - _N)
    src3 = jnp.concatenate(
        [src, jnp.full((pad,), _ZROW, jnp.int32)]).reshape(_NW, n_chunks, _CHUNK)
    dst3 = jnp.concatenate(
        [dst, pad_dst]).reshape(_NW, n_chunks, _CHUNK)

    z1 = jnp.zeros((_NPAD,), f32)
    z8 = jnp.zeros((_NPAD, 8), f32)
    z80 = jnp.zeros((_NPAD, 80), f32)

    x_p = _pad2(x, _NPAD, 128)
    w1_p = _pad2(W1, 128, 80)
    w2_p = _pad2(W2, 80, 8)
    w3_p = _pad2(W3, 8, 8)
    wc_p = _pad2(Wc, 8, 12)
    b1_p = jnp.pad(b1, (0, 80 - b1.shape[0]))[None, :]
    b2_p = jnp.pad(b2, (0, 8 - b2.shape[0]))[None, :]
    b3_p = jnp.pad(b3, (0, 8 - b3.shape[0]))[None, :]
    bc_p = bc[None, :]

    degp = _sc_degree(n_chunks)(dst3, z1)
    y1, dinv = _tc_pre(x_p, w1_p, degp.T)

    p1 = _sc_aggregate(80, n_chunks)(y1, src3, dst3, z80)
    y2 = _tc_mid(p1[0], p1[1], y1, dinv, b1_p, w2_p)

    p2 = _sc_aggregate(8, n_chunks)(y2, src3, dst3, z8)
    y3 = _tc_mid(p2[0], p2[1], y2, dinv, b2_p, w3_p)

    p3 = _sc_aggregate(8, n_chunks)(y3, src3, dst3, z8)
    h_p, out_p = _tc_final(p3[0], p3[1], y3, dinv, b3_p, wc_p, bc_p)

    return out_p[:_N], h_p[:_N, :4]


# speedup vs baseline: 1.9670x; 1.1107x over previous
import functools

import jax
import jax.numpy as jnp
from jax import lax
from jax.experimental import pallas as pl
from jax.experimental.pallas import tpu as pltpu
from jax.experimental.pallas import tpu_sc as plsc

_N = 10000
_NPAD = 10240
_ZROW = _N
_NC, _NS = 2, 16
_NW = _NC * _NS
_CHUNK = 128
_RPT = _NPAD // _NS


def _sc_mesh():
    return plsc.VectorSubcoreMesh(core_axis_name="c", subcore_axis_name="s")


_SC_PARAMS = pltpu.CompilerParams(use_tc_tiling_on_sc=False)
_SC_PARAMS_NARROW = pltpu.CompilerParams(use_tc_tiling_on_sc=False,
                                         needs_layout_passes=False)


def _sc_degree(n_chunks):
    L = 16

    @functools.partial(
        pl.kernel,
        mesh=_sc_mesh(),
        out_type=jax.ShapeDtypeStruct((_NC, _NPAD), jnp.float32),
        scratch_types=[
            pltpu.VMEM((n_chunks, _CHUNK), jnp.int32),
            pltpu.VMEM((_CHUNK,), jnp.float32),
            pltpu.VMEM_SHARED((_NPAD,), jnp.float32),
        ],
        compiler_params=_SC_PARAMS,
    )
    def k(dst_hbm, zeros_hbm, out_hbm, dst_v, ones_v, acc):
        c = lax.axis_index("c")
        s = lax.axis_index("s")
        wid = c * _NS + s
        pltpu.sync_copy(dst_hbm.at[wid], dst_v)
        one = lax.full((L,), 1.0, jnp.float32)
        for g in range(_CHUNK // L):
            ones_v[pl.ds(g * L, L)] = one
        pltpu.sync_copy(zeros_hbm.at[pl.ds(s * _RPT, _RPT)],
                        acc.at[pl.ds(s * _RPT, _RPT)])
        plsc.subcore_barrier()

        def body(i, carry):
            pltpu.sync_copy(ones_v, acc.at[dst_v.at[i]], add=True)
            return carry

        lax.fori_loop(0, n_chunks, body, 0)
        plsc.subcore_barrier()
        pltpu.sync_copy(acc.at[pl.ds(s * _RPT, _RPT)],
                        out_hbm.at[c, pl.ds(s * _RPT, _RPT)])

    return k


_NBUF = 4


def _sc_aggregate(d_feat, n_chunks):
    @functools.partial(
        pl.kernel,
        mesh=_sc_mesh(),
        out_type=jax.ShapeDtypeStruct((_NC, _NPAD, d_feat), jnp.float32),
        scratch_types=[
            pltpu.VMEM((n_chunks, _CHUNK), jnp.int32),
            pltpu.VMEM((n_chunks, _CHUNK), jnp.int32),
            pltpu.VMEM((_CHUNK, d_feat), jnp.float32),
            pltpu.VMEM_SHARED((_NPAD, d_feat), jnp.float32),
            pltpu.SemaphoreType.DMA,
        ],
        compiler_params=_SC_PARAMS,
    )
    def k(y_hbm, src_hbm, dst_hbm, zeros_hbm, out_hbm,
          src_v, dst_v, rows_v, acc, sem):
        c = lax.axis_index("c")
        s = lax.axis_index("s")
        wid = c * _NS + s
        pltpu.sync_copy(src_hbm.at[wid], src_v)
        pltpu.sync_copy(dst_hbm.at[wid], dst_v)
        pltpu.sync_copy(zeros_hbm.at[pl.ds(s * _RPT, _RPT)],
                        acc.at[pl.ds(s * _RPT, _RPT)])
        plsc.subcore_barrier()

        def body(i, carry):
            pltpu.async_copy(y_hbm.at[src_v.at[i]], rows_v, sem).wait()
            pltpu.sync_copy(rows_v, acc.at[dst_v.at[i]], add=True)
            return carry

        lax.fori_loop(0, n_chunks, body, 0)
        plsc.subcore_barrier()
        pltpu.sync_copy(acc.at[pl.ds(s * _RPT, _RPT)],
                        out_hbm.at[c, pl.ds(s * _RPT, _RPT)])

    return k


def _sc_aggregate_narrow(d_feat, n_chunks):
    L = 16

    @functools.partial(
        pl.kernel,
        mesh=_sc_mesh(),
        out_type=jax.ShapeDtypeStruct((_NC, _NPAD, d_feat), jnp.float32),
        scratch_types=[
            pltpu.VMEM((n_chunks, _CHUNK), jnp.int32),
            pltpu.VMEM((n_chunks, _CHUNK), jnp.int32),
            pltpu.VMEM((_NPAD, d_feat), jnp.float32),
            pltpu.VMEM((_CHUNK, d_feat), jnp.float32),
            pltpu.VMEM_SHARED((_NPAD, d_feat), jnp.float32),
        ],
        compiler_params=_SC_PARAMS_NARROW,
    )
    def k(y_hbm, src_hbm, dst_hbm, zeros_hbm, out_hbm,
          src_v, dst_v, y_v, rows_v, acc):
        c = lax.axis_index("c")
        s = lax.axis_index("s")
        wid = c * _NS + s
        pltpu.sync_copy(src_hbm.at[wid], src_v)
        pltpu.sync_copy(dst_hbm.at[wid], dst_v)
        pltpu.sync_copy(y_hbm, y_v)
        pltpu.sync_copy(zeros_hbm.at[pl.ds(s * _RPT, _RPT)],
                        acc.at[pl.ds(s * _RPT, _RPT)])
        plsc.subcore_barrier()

        lane = lax.iota(jnp.int32, L)
        cols = [lax.full((L,), cc, jnp.int32) for cc in range(d_feat)]
        ids = [lane + (g * L) for g in range(_CHUNK // L)]

        def body(i, carry):
            for g in range(_CHUNK // L):
                src16 = src_v[i, pl.ds(g * L, L)]
                for cc in range(d_feat):
                    vals = plsc.load_gather(y_v, [src16, cols[cc]])
                    plsc.store_scatter(rows_v, [ids[g], cols[cc]], vals)
            pltpu.sync_copy(rows_v, acc.at[dst_v.at[i]], add=True)
            return carry

        lax.fori_loop(0, n_chunks, body, 0)
        plsc.subcore_barrier()
        pltpu.sync_copy(acc.at[pl.ds(s * _RPT, _RPT)],
                        out_hbm.at[c, pl.ds(s * _RPT, _RPT)])

    return k


def _tc_pre(x_p, w1_p, degp_t):
    def body(x_ref, w_ref, dp_ref, y_ref, dinv_ref):
        dinv = lax.rsqrt(dp_ref[:, 0:1] + dp_ref[:, 1:2] + 1.0)
        xw = jnp.dot(x_ref[...], w_ref[...],
                     preferred_element_type=jnp.float32)
        y_ref[...] = xw * dinv
        dinv_ref[...] = dinv

    return pl.pallas_call(
        body,
        out_shape=[
            jax.ShapeDtypeStruct((_NPAD, w1_p.shape[1]), jnp.float32),
            jax.ShapeDtypeStruct((_NPAD, 1), jnp.float32),
        ],
    )(x_p, w1_p, degp_t)


def _tc_mid(p0, p1, y_prev, dinv, b_p, w_p):
    def body(p0_ref, p1_ref, yp_ref, dinv_ref, b_ref, w_ref, yn_ref):
        tot = p0_ref[...] + p1_ref[...] + yp_ref[...]
        h = jnp.tanh(dinv_ref[...] * tot + b_ref[...])
        rows = lax.broadcasted_iota(jnp.int32, (_NPAD, 1), 0)
        h = jnp.where(rows < _N, h, 0.0)
        yn_ref[...] = dinv_ref[...] * jnp.dot(
            h, w_ref[...], preferred_element_type=jnp.float32)

    return pl.pallas_call(
        body,
        out_shape=jax.ShapeDtypeStruct((_NPAD, w_p.shape[1]), jnp.float32),
    )(p0, p1, y_prev, dinv, b_p, w_p)


def _tc_final(p0, p1, y_prev, dinv, b_p, wc_p, bc_p):
    def body(p0_ref, p1_ref, yp_ref, dinv_ref, b_ref, wc_ref, bc_ref,
             h_ref, out_ref):
        tot = p0_ref[...] + p1_ref[...] + yp_ref[...]
        h = jnp.tanh(dinv_ref[...] * tot + b_ref[...])
        rows = lax.broadcasted_iota(jnp.int32, (_NPAD, 1), 0)
        h = jnp.where(rows < _N, h, 0.0)
        h_ref[...] = h
        out_ref[...] = jnp.dot(
            h, wc_ref[...], preferred_element_type=jnp.float32) + bc_ref[...]

    return pl.pallas_call(
        body,
        out_shape=[
            jax.ShapeDtypeStruct((_NPAD, b_p.shape[1]), jnp.float32),
            jax.ShapeDtypeStruct((_NPAD, wc_p.shape[1]), jnp.float32),
        ],
    )(p0, p1, y_prev, dinv, b_p, wc_p, bc_p)


def _pad2(a, rows, cols):
    return jnp.pad(a, ((0, rows - a.shape[0]), (0, cols - a.shape[1])))


def kernel(x, edge_index, W1, b1, W2, b2, W3, b3, Wc, bc):
    f32 = jnp.float32
    src = edge_index[0].astype(jnp.int32)
    dst = edge_index[1].astype(jnp.int32)
    n_edges = src.shape[0]
    n_chunks = -(-n_edges // (_NW * _CHUNK))
    e_pad = _NW * _CHUNK * n_chunks
    pad = e_pad - n_edges

    pad_dst = _N + jnp.arange(pad, dtype=jnp.int32) % (_NPAD - _N)
    src3 = jnp.concatenate(
        [src, jnp.full((pad,), _ZROW, jnp.int32)]).reshape(_NW, n_chunks, _CHUNK)
    dst3 = jnp.concatenate(
        [dst, pad_dst]).reshape(_NW, n_chunks, _CHUNK)

    z1 = jnp.zeros((_NPAD,), f32)
    z8 = jnp.zeros((_NPAD, 8), f32)
    z80 = jnp.zeros((_NPAD, 80), f32)

    x_p = _pad2(x, _NPAD, 128)
    w1_p = _pad2(W1, 128, 80)
    w2_p = _pad2(W2, 80, 8)
    w3_p = _pad2(W3, 8, 8)
    wc_p = _pad2(Wc, 8, 12)
    b1_p = jnp.pad(b1, (0, 80 - b1.shape[0]))[None, :]
    b2_p = jnp.pad(b2, (0, 8 - b2.shape[0]))[None, :]
    b3_p = jnp.pad(b3, (0, 8 - b3.shape[0]))[None, :]
    bc_p = bc[None, :]

    degp = _sc_degree(n_chunks)(dst3, z1)
    y1, dinv = _tc_pre(x_p, w1_p, degp.T)

    p1 = _sc_aggregate(80, n_chunks)(y1, src3, dst3, z80)
    y2 = _tc_mid(p1[0], p1[1], y1, dinv, b1_p, w2_p)

    p2 = _sc_aggregate_narrow(8, n_chunks)(y2, src3, dst3, z8)
    y3 = _tc_mid(p2[0], p2[1], y2, dinv, b2_p, w3_p)

    p3 = _sc_aggregate_narrow(8, n_chunks)(y3, src3, dst3, z8)
    h_p, out_p = _tc_final(p3[0], p3[1], y3, dinv, b3_p, wc_p, bc_p)

    return out_p[:_N], h_p[:_N, :4]
